# baseline (speedup 1.0000x reference)
import functools

import jax
import jax.numpy as jnp
from jax import lax
from jax.experimental import pallas as pl
from jax.experimental.pallas import tpu as pltpu
from jax.experimental.pallas import tpu_sc as plsc

_B = 16384
_F = 26
_VOCAB = 100000
_D = 32

_NC = 2
_NS = 16
_NW = _NC * _NS
_NB = _B // _NW


@functools.partial(
    pl.kernel,
    out_type=jax.ShapeDtypeStruct((_B, _F, _D), jnp.float32),
    mesh=plsc.VectorSubcoreMesh(core_axis_name="c", subcore_axis_name="s"),
    scratch_types=[
        pltpu.VMEM((_NB, _F), jnp.int32),
        pltpu.VMEM((_NB,), jnp.int32),
        pltpu.VMEM((_NB,), jnp.int32),
        pltpu.VMEM((_NB, _D), jnp.float32),
        pltpu.VMEM((_NB, _D), jnp.float32),
        pltpu.SemaphoreType.DMA,
        pltpu.SemaphoreType.DMA,
        pltpu.SemaphoreType.DMA,
        pltpu.SemaphoreType.DMA,
    ],
    compiler_params=pltpu.CompilerParams(
        use_tc_tiling_on_sc=False, needs_layout_passes=False
    ),
)
def _embed_kernel(x_in, tables_in, out_3d, xblk, i0, i1, r0, r1, g0, g1, w0, w1):
    wid = lax.axis_index("s") * _NC + lax.axis_index("c")
    b0 = wid * _NB
    idx = (i0, i1)
    bufs = (r0, r1)
    gsem = (g0, g1)
    wsem = (w0, w1)
    lanes = lax.iota(jnp.int32, 16)

    pltpu.sync_copy(x_in.at[pl.ds(b0, _NB)], xblk)

    def stage_column(f):
        col = jnp.full((16,), f, dtype=jnp.int32)
        dst = idx[f % 2]

        def vec_body(j, carry):
            rows = j * 16 + lanes
            dst[pl.ds(j * 16, 16)] = plsc.load_gather(xblk, [rows, col])
            return carry

        lax.fori_loop(0, _NB // 16, vec_body, 0)

    def issue_gather(f):
        return pltpu.async_copy(
            tables_in.at[f].at[idx[f % 2]], bufs[f % 2], gsem[f % 2]
        )

    def issue_write(f):
        return pltpu.async_copy(
            bufs[f % 2], out_3d.at[pl.ds(b0, _NB), f], wsem[f % 2]
        )

    stage_column(0)
    gathers = [issue_gather(0)]
    writes = [None, None]
    for f in range(_F):
        p = f % 2
        if f + 1 < _F:
            stage_column(f + 1)
            if writes[1 - p] is not None:
                writes[1 - p].wait()
            gathers.append(issue_gather(f + 1))
        gathers[f].wait()
        writes[p] = issue_write(f)
    writes[0].wait()
    writes[1].wait()


def kernel(x, tables):
    return _embed_kernel(x, tables)


# speedup vs baseline: 3.4824x; 3.4824x over previous
import functools

import jax
import jax.numpy as jnp
from jax import lax
from jax.experimental import pallas as pl
from jax.experimental.pallas import tpu as pltpu
from jax.experimental.pallas import tpu_sc as plsc

_B = 16384
_F = 26
_VOCAB = 100000
_D = 32

_NC = 2
_NS = 16
_NW = _NC * _NS
_CH = 4096
_NCH = _B // _CH


@functools.partial(
    pl.kernel,
    out_type=jax.ShapeDtypeStruct((_F, _D, _B), jnp.float32),
    mesh=plsc.VectorSubcoreMesh(core_axis_name="c", subcore_axis_name="s"),
    scratch_types=[
        pltpu.VMEM((_VOCAB,), jnp.float32),
        pltpu.VMEM((_CH,), jnp.int32),
        pltpu.VMEM((_CH,), jnp.float32),
    ],
    compiler_params=pltpu.CompilerParams(needs_layout_passes=False),
)
def _embed_kernel(xt_in, tabt_in, out_t, tfd, xbuf, obuf):
    d = lax.axis_index("s") * _NC + lax.axis_index("c")

    def feature_body(f, carry):
        pltpu.sync_copy(tabt_in.at[f, d], tfd)

        def chunk_body(c, carry2):
            b0 = c * _CH
            pltpu.sync_copy(xt_in.at[f, pl.ds(b0, _CH)], xbuf)

            def vec_body(j, carry3):
                v = xbuf[pl.ds(j * 16, 16)]
                obuf[pl.ds(j * 16, 16)] = plsc.load_gather(tfd, [v])
                return carry3

            lax.fori_loop(0, _CH // 16, vec_body, 0)
            pltpu.sync_copy(obuf, out_t.at[f, d, pl.ds(b0, _CH)])
            return carry2

        lax.fori_loop(0, _NCH, chunk_body, 0)
        return carry

    lax.fori_loop(0, _F, feature_body, 0)


def kernel(x, tables):
    out_t = _embed_kernel(x.T, tables.transpose(0, 2, 1))
    return out_t.transpose(2, 0, 1)


# speedup vs baseline: 6.3250x; 1.8163x over previous
import functools

import jax
import jax.numpy as jnp
from jax import lax
from jax.experimental import pallas as pl
from jax.experimental.pallas import tpu as pltpu
from jax.experimental.pallas import tpu_sc as plsc

_B = 16384
_F = 26
_VOCAB = 100000
_D = 32

_NC = 2
_NS = 16
_NW = _NC * _NS
_CH = 4096
_NCH = _B // _CH


@functools.partial(
    pl.kernel,
    out_type=jax.ShapeDtypeStruct((_F, _D, _B), jnp.float32),
    mesh=plsc.VectorSubcoreMesh(core_axis_name="c", subcore_axis_name="s"),
    scratch_types=[
        pltpu.VMEM((_VOCAB,), jnp.float32),
        pltpu.VMEM((_CH,), jnp.int32),
        pltpu.VMEM((_CH,), jnp.int32),
        pltpu.VMEM((_CH,), jnp.float32),
        pltpu.VMEM((_CH,), jnp.float32),
        pltpu.SemaphoreType.DMA,
        pltpu.SemaphoreType.DMA,
        pltpu.SemaphoreType.DMA,
        pltpu.SemaphoreType.DMA,
    ],
    compiler_params=pltpu.CompilerParams(needs_layout_passes=False),
)
def _embed_kernel(xt_in, tabt_in, out_t, tfd, xb0, xb1, ob0, ob1, sx0, sx1, so0, so1):
    d = lax.axis_index("s") * _NC + lax.axis_index("c")
    xb = (xb0, xb1)
    ob = (ob0, ob1)
    sx = (sx0, sx1)
    so = (so0, so1)

    def x_copy(f, c):
        return pltpu.make_async_copy(
            xt_in.at[f, pl.ds(c * _CH, _CH)], xb[c % 2], sx[c % 2]
        )

    def o_copy(f, c):
        return pltpu.make_async_copy(
            ob[c % 2], out_t.at[f, d, pl.ds(c * _CH, _CH)], so[c % 2]
        )

    def feature_body(f, carry):
        x_copy(f, 0).start()
        pltpu.sync_copy(tabt_in.at[f, d], tfd)

        for c in range(_NCH):
            if c + 1 < _NCH:
                x_copy(f, c + 1).start()
            x_copy(f, c).wait()
            if c >= 2:
                o_copy(f, c - 2).wait()
            dst = ob[c % 2]
            src = xb[c % 2]

            @plsc.parallel_loop(0, _CH, step=16, unroll=8)
            def gather_body(i):
                dst[pl.ds(i, 16)] = plsc.load_gather(tfd, [src[pl.ds(i, 16)]])

            o_copy(f, c).start()
        o_copy(f, _NCH - 2).wait()
        o_copy(f, _NCH - 1).wait()
        return carry

    lax.fori_loop(0, _F, feature_body, 0)


def kernel(x, tables):
    out_t = _embed_kernel(x.T, tables.transpose(0, 2, 1))
    return out_t.transpose(2, 0, 1)


# speedup vs baseline: 6.3398x; 1.0023x over previous
import functools

import jax
import jax.numpy as jnp
from jax import lax
from jax.experimental import pallas as pl
from jax.experimental.pallas import tpu as pltpu
from jax.experimental.pallas import tpu_sc as plsc

_B = 16384
_F = 26
_VOCAB = 100000
_D = 32

_NC = 2
_NS = 16
_NW = _NC * _NS
_CH = 4096
_NCH = _B // _CH


@functools.partial(
    pl.kernel,
    out_type=jax.ShapeDtypeStruct((_F, _D, _B), jnp.float32),
    mesh=plsc.VectorSubcoreMesh(core_axis_name="c", subcore_axis_name="s"),
    scratch_types=[
        pltpu.VMEM((_VOCAB,), jnp.float32),
        pltpu.VMEM((_CH,), jnp.int32),
        pltpu.VMEM((_CH,), jnp.int32),
        pltpu.VMEM((_CH,), jnp.float32),
        pltpu.VMEM((_CH,), jnp.float32),
        pltpu.SemaphoreType.DMA,
        pltpu.SemaphoreType.DMA,
        pltpu.SemaphoreType.DMA,
        pltpu.SemaphoreType.DMA,
    ],
    compiler_params=pltpu.CompilerParams(needs_layout_passes=False),
)
def _embed_kernel(xt_in, tabt_in, out_t, tfd, xb0, xb1, ob0, ob1, sx0, sx1, so0, so1):
    d = lax.axis_index("s") * _NC + lax.axis_index("c")
    xb = (xb0, xb1)
    ob = (ob0, ob1)
    sx = (sx0, sx1)
    so = (so0, so1)

    def x_copy(f, c):
        return pltpu.make_async_copy(
            xt_in.at[f, pl.ds(c * _CH, _CH)], xb[c % 2], sx[c % 2]
        )

    def o_copy(f, c):
        return pltpu.make_async_copy(
            ob[c % 2], out_t.at[f, d, pl.ds(c * _CH, _CH)], so[c % 2]
        )

    def feature_body(f, carry):
        x_copy(f, 0).start()
        pltpu.sync_copy(tabt_in.at[f, d], tfd)

        for c in range(_NCH):
            if c + 1 < _NCH:
                x_copy(f, c + 1).start()
            x_copy(f, c).wait()
            if c >= 2:
                o_copy(f, c - 2).wait()
            dst = ob[c % 2]
            src = xb[c % 2]

            @plsc.parallel_loop(0, _CH, step=16, unroll=16)
            def gather_body(i):
                dst[pl.ds(i, 16)] = plsc.load_gather(tfd, [src[pl.ds(i, 16)]])

            o_copy(f, c).start()
        o_copy(f, _NCH - 2).wait()
        o_copy(f, _NCH - 1).wait()
        return carry

    lax.fori_loop(0, _F, feature_body, 0)


def kernel(x, tables):
    out_t = _embed_kernel(x.T, tables.transpose(0, 2, 1))
    return out_t.transpose(2, 0, 1)


# speedup vs baseline: 6.3426x; 1.0004x over previous
import functools

import jax
import jax.numpy as jnp
from jax import lax
from jax.experimental import pallas as pl
from jax.experimental.pallas import tpu as pltpu
from jax.experimental.pallas import tpu_sc as plsc

_B = 16384
_F = 26
_VOCAB = 100000
_D = 32

_NC = 2
_NS = 16
_NW = _NC * _NS
_CH = 4096
_NCH = _B // _CH


@functools.partial(
    pl.kernel,
    out_type=jax.ShapeDtypeStruct((_F, _D, _B), jnp.float32),
    mesh=plsc.VectorSubcoreMesh(core_axis_name="c", subcore_axis_name="s"),
    scratch_types=[
        pltpu.VMEM((_VOCAB,), jnp.float32),
        pltpu.VMEM((_CH,), jnp.int32),
        pltpu.VMEM((_CH,), jnp.int32),
        pltpu.VMEM((_CH,), jnp.float32),
        pltpu.VMEM((_CH,), jnp.float32),
        pltpu.SemaphoreType.DMA,
        pltpu.SemaphoreType.DMA,
        pltpu.SemaphoreType.DMA,
        pltpu.SemaphoreType.DMA,
    ],
    compiler_params=pltpu.CompilerParams(needs_layout_passes=False),
)
def _embed_kernel(xt_in, tabt_in, out_t, tfd, xb0, xb1, ob0, ob1, sx0, sx1, so0, so1):
    d = lax.axis_index("c") * _NS + lax.axis_index("s")
    xb = (xb0, xb1)
    ob = (ob0, ob1)
    sx = (sx0, sx1)
    so = (so0, so1)

    def x_copy(f, c):
        return pltpu.make_async_copy(
            xt_in.at[f, pl.ds(c * _CH, _CH)], xb[c % 2], sx[c % 2]
        )

    def o_copy(f, c):
        return pltpu.make_async_copy(
            ob[c % 2], out_t.at[f, d, pl.ds(c * _CH, _CH)], so[c % 2]
        )

    def feature_body(f, carry):
        x_copy(f, 0).start()
        pltpu.sync_copy(tabt_in.at[f, d], tfd)

        for c in range(_NCH):
            if c + 1 < _NCH:
                x_copy(f, c + 1).start()
            x_copy(f, c).wait()
            if c >= 2:
                o_copy(f, c - 2).wait()
            dst = ob[c % 2]
            src = xb[c % 2]

            @plsc.parallel_loop(0, _CH, step=16, unroll=16)
            def gather_body(i):
                dst[pl.ds(i, 16)] = plsc.load_gather(tfd, [src[pl.ds(i, 16)]])

            o_copy(f, c).start()
        o_copy(f, _NCH - 2).wait()
        o_copy(f, _NCH - 1).wait()
        return carry

    lax.fori_loop(0, _F, feature_body, 0)


def kernel(x, tables):
    out_t = _embed_kernel(x.T, tables.transpose(0, 2, 1))
    return out_t.transpose(2, 0, 1)


# speedup vs baseline: 6.9807x; 1.1006x over previous
import functools

import jax
import jax.numpy as jnp
from jax import lax
from jax.experimental import pallas as pl
from jax.experimental.pallas import tpu as pltpu
from jax.experimental.pallas import tpu_sc as plsc

_B = 16384
_F = 26
_VOCAB = 100000
_D = 32

_NC = 2
_NS = 16
_NW = _NC * _NS
_CH = 4096
_NCH = _B // _CH


@functools.partial(
    pl.kernel,
    out_type=jax.ShapeDtypeStruct((_F, _D, _B), jnp.float32),
    mesh=plsc.VectorSubcoreMesh(core_axis_name="c", subcore_axis_name="s"),
    scratch_types=[
        pltpu.VMEM((_VOCAB,), jnp.float32),
        pltpu.VMEM((_CH,), jnp.int32),
        pltpu.VMEM((_CH,), jnp.int32),
        pltpu.VMEM((_CH,), jnp.float32),
        pltpu.VMEM((_CH,), jnp.float32),
        pltpu.VMEM_SHARED((2, _B), jnp.int32),
        pltpu.SemaphoreType.DMA,
        pltpu.SemaphoreType.DMA,
        pltpu.SemaphoreType.DMA,
        pltpu.SemaphoreType.DMA,
    ],
    compiler_params=pltpu.CompilerParams(needs_layout_passes=False),
)
def _embed_kernel(
    xt_in, tabt_in, out_t, tfd, xb0, xb1, ob0, ob1, xsh, sx0, sx1, so0, so1
):
    s = lax.axis_index("s")
    d = lax.axis_index("c") * _NS + s
    xb = (xb0, xb1)
    ob = (ob0, ob1)
    sx = (sx0, sx1)
    so = (so0, so1)

    def x_copy(f, c):
        return pltpu.make_async_copy(
            xsh.at[f % 2, pl.ds(c * _CH, _CH)], xb[c % 2], sx[c % 2]
        )

    def o_copy(f, c):
        return pltpu.make_async_copy(
            ob[c % 2], out_t.at[f, d, pl.ds(c * _CH, _CH)], so[c % 2]
        )

    @pl.when(s == 0)
    def _():
        pltpu.sync_copy(xt_in.at[0], xsh.at[0])

    plsc.subcore_barrier()

    def feature_body(f, carry):
        @pl.when(s == 0)
        def _():
            nf = lax.min(f + 1, _F - 1)
            pltpu.sync_copy(xt_in.at[nf], xsh.at[(f + 1) % 2])

        x_copy(f, 0).start()
        pltpu.sync_copy(tabt_in.at[f, d], tfd)

        for c in range(_NCH):
            if c + 1 < _NCH:
                x_copy(f, c + 1).start()
            x_copy(f, c).wait()
            if c >= 2:
                o_copy(f, c - 2).wait()
            dst = ob[c % 2]
            src = xb[c % 2]

            @plsc.parallel_loop(0, _CH, step=16, unroll=8)
            def gather_body(i):
                dst[pl.ds(i, 16)] = plsc.load_gather(tfd, [src[pl.ds(i, 16)]])

            o_copy(f, c).start()
        o_copy(f, _NCH - 2).wait()
        o_copy(f, _NCH - 1).wait()
        plsc.subcore_barrier()
        return carry

    lax.fori_loop(0, _F, feature_body, 0)


def kernel(x, tables):
    out_t = _embed_kernel(x.T, tables.transpose(0, 2, 1))
    return out_t.transpose(2, 0, 1)


# speedup vs baseline: 7.6037x; 1.0892x over previous
import functools

import jax
import jax.numpy as jnp
from jax import lax
from jax.experimental import pallas as pl
from jax.experimental.pallas import tpu as pltpu
from jax.experimental.pallas import tpu_sc as plsc

_B = 16384
_F = 26
_VOCAB = 100000
_D = 32

_NC = 2
_NS = 16
_NW = _NC * _NS
_CH = 4096
_NCH = _B // _CH


@functools.partial(
    pl.kernel,
    out_type=jax.ShapeDtypeStruct((_F, _D, _B), jnp.float32),
    mesh=plsc.VectorSubcoreMesh(core_axis_name="c", subcore_axis_name="s"),
    scratch_types=[
        pltpu.VMEM((_VOCAB,), jnp.float32),
        pltpu.VMEM((_CH,), jnp.int32),
        pltpu.VMEM((_CH,), jnp.int32),
        pltpu.VMEM((_CH,), jnp.float32),
        pltpu.VMEM((_CH,), jnp.float32),
        pltpu.VMEM_SHARED((2, _B), jnp.int32),
        pltpu.SemaphoreType.DMA,
        pltpu.SemaphoreType.DMA,
        pltpu.SemaphoreType.DMA,
        pltpu.SemaphoreType.DMA,
        pltpu.SemaphoreType.DMA,
    ],
    compiler_params=pltpu.CompilerParams(needs_layout_passes=False),
)
def _embed_kernel(
    xt_in, tabt_in, out_t, tfd, xb0, xb1, ob0, ob1, xsh, sx0, sx1, so0, so1, sxp
):
    s = lax.axis_index("s")
    d = lax.axis_index("c") * _NS + s
    xb = (xb0, xb1)
    ob = (ob0, ob1)
    sx = (sx0, sx1)
    so = (so0, so1)

    def x_copy(f, c):
        return pltpu.make_async_copy(
            xsh.at[f % 2, pl.ds(c * _CH, _CH)], xb[c % 2], sx[c % 2]
        )

    def o_copy(f, c):
        return pltpu.make_async_copy(
            ob[c % 2], out_t.at[f, d, pl.ds(c * _CH, _CH)], so[c % 2]
        )

    @pl.when(s == 0)
    def _():
        pltpu.sync_copy(xt_in.at[0], xsh.at[0])

    plsc.subcore_barrier()

    def feature_body(f, carry):
        def x_prefetch():
            nf = lax.min(f + 1, _F - 1)
            return pltpu.make_async_copy(
                xt_in.at[nf], xsh.at[(f + 1) % 2], sxp
            )

        @pl.when(s == 0)
        def _():
            x_prefetch().start()

        x_copy(f, 0).start()
        pltpu.sync_copy(tabt_in.at[f, d], tfd)

        for c in range(_NCH):
            if c + 1 < _NCH:
                x_copy(f, c + 1).start()
            x_copy(f, c).wait()
            if c >= 2:
                o_copy(f, c - 2).wait()
            dst = ob[c % 2]
            src = xb[c % 2]

            @plsc.parallel_loop(0, _CH, step=16, unroll=8)
            def gather_body(i):
                dst[pl.ds(i, 16)] = plsc.load_gather(tfd, [src[pl.ds(i, 16)]])

            o_copy(f, c).start()
        o_copy(f, _NCH - 2).wait()
        o_copy(f, _NCH - 1).wait()

        @pl.when(s == 0)
        def _():
            x_prefetch().wait()

        plsc.subcore_barrier()
        return carry

    lax.fori_loop(0, _F, feature_body, 0)


def kernel(x, tables):
    out_t = _embed_kernel(x.T, tables.transpose(0, 2, 1))
    return out_t.transpose(2, 0, 1)
